# baseline (speedup 1.0000x reference)
import functools

import jax
import jax.numpy as jnp
from jax import lax
from jax.experimental import pallas as pl
from jax.experimental.pallas import tpu as pltpu
from jax.experimental.pallas import tpu_sc as plsc

_B, _C, _H, _W = 64, 3, 256, 256
_NC, _NS, _L = 2, 16, 16
_NW = _NC * _NS
_RPW = _B // _NW
_RCHUNK = 64
_NCH = _H // _RCHUNK

_mesh = plsc.VectorSubcoreMesh(core_axis_name="c", subcore_axis_name="s")


@functools.partial(
    pl.kernel,
    mesh=_mesh,
    out_type=jax.ShapeDtypeStruct((_B, _C, _H, _W), jnp.float32),
    scratch_types=[
        pltpu.VMEM((_B,), jnp.int32),
        pltpu.VMEM((1000,), jnp.float32),
        pltpu.VMEM((1000,), jnp.float32),
        pltpu.VMEM((_RCHUNK, _W), jnp.float32),
        pltpu.VMEM((_RCHUNK, _W), jnp.float32),
        pltpu.VMEM((_RCHUNK, _W), jnp.float32),
    ],
    compiler_params=pltpu.CompilerParams(needs_layout_passes=False),
)
def _sc_qsample(x0_hbm, t_hbm, n_hbm, a_hbm, s_hbm, out_hbm,
                t_v, a_tbl, s_tbl, xb, nb, ob):
    wid = lax.axis_index("s") * _NC + lax.axis_index("c")
    pltpu.sync_copy(t_hbm, t_v)
    pltpu.sync_copy(a_hbm, a_tbl)
    pltpu.sync_copy(s_hbm, s_tbl)

    for k in range(_RPW):
        g = wid * _RPW + k
        gidx = jnp.full((_L,), g, jnp.int32)
        tt = plsc.load_gather(t_v, [gidx])
        av = plsc.load_gather(a_tbl, [tt])
        sv = plsc.load_gather(s_tbl, [tt])

        for c in range(_C):
            for rr in range(_NCH):
                rows = pl.ds(rr * _RCHUNK, _RCHUNK)
                pltpu.sync_copy(x0_hbm.at[g, c, rows], xb)
                pltpu.sync_copy(n_hbm.at[g, c, rows], nb)

                def _row(i, carry):
                    for j in range(_W // _L):
                        cols = pl.ds(j * _L, _L)
                        ob[i, cols] = av * xb[i, cols] + sv * nb[i, cols]
                    return carry

                lax.fori_loop(0, _RCHUNK, _row, 0)
                pltpu.sync_copy(ob, out_hbm.at[g, c, rows])


def kernel(x0, t, noise, sqrt_alphas_bar, sqrt_one_minus_alphas_bar):
    xt = _sc_qsample(
        x0,
        t.astype(jnp.int32),
        noise,
        sqrt_alphas_bar,
        sqrt_one_minus_alphas_bar,
    )
    return xt, noise


# speedup vs baseline: 1.4750x; 1.4750x over previous
import functools

import jax
import jax.numpy as jnp
from jax import lax
from jax.experimental import pallas as pl
from jax.experimental.pallas import tpu as pltpu
from jax.experimental.pallas import tpu_sc as plsc

_B, _C, _H, _W = 64, 3, 256, 256
_NC, _NS, _L = 2, 16, 16
_NW = _NC * _NS
_RPW = _B // _NW
_RCHUNK = 64
_NCH = _H // _RCHUNK
_NCHUNKS = _RPW * _C * _NCH

_mesh = plsc.VectorSubcoreMesh(core_axis_name="c", subcore_axis_name="s")


def _chunk_coords(gch):
    k = gch // (_C * _NCH)
    rem = gch % (_C * _NCH)
    return k, rem // _NCH, pl.ds((rem % _NCH) * _RCHUNK, _RCHUNK)


@functools.partial(
    pl.kernel,
    mesh=_mesh,
    out_type=jax.ShapeDtypeStruct((_B, _C, _H, _W), jnp.float32),
    scratch_types=[
        pltpu.VMEM((_B,), jnp.int32),
        pltpu.VMEM((1000,), jnp.float32),
        pltpu.VMEM((1000,), jnp.float32),
    ]
    + [pltpu.VMEM((_RCHUNK, _W), jnp.float32)] * 6
    + [pltpu.SemaphoreType.DMA] * 6,
    compiler_params=pltpu.CompilerParams(needs_layout_passes=False),
)
def _sc_qsample(x0_hbm, t_hbm, n_hbm, a_hbm, s_hbm, out_hbm,
                t_v, a_tbl, s_tbl, xb0, xb1, nb0, nb1, ob0, ob1,
                sx0, sx1, sn0, sn1, so0, so1):
    wid = lax.axis_index("s") * _NC + lax.axis_index("c")
    pltpu.sync_copy(t_hbm, t_v)
    pltpu.sync_copy(a_hbm, a_tbl)
    pltpu.sync_copy(s_hbm, s_tbl)

    coefs = []
    for k in range(_RPW):
        gidx = jnp.full((_L,), wid * _RPW + k, jnp.int32)
        tt = plsc.load_gather(t_v, [gidx])
        coefs.append((plsc.load_gather(a_tbl, [tt]),
                      plsc.load_gather(s_tbl, [tt])))

    xbs, nbs, obs = (xb0, xb1), (nb0, nb1), (ob0, ob1)
    sxs, sns, sos = (sx0, sx1), (sn0, sn1), (so0, so1)

    in_handles = [None, None]
    out_handles = [None, None]

    k0, c0, rows0 = _chunk_coords(0)
    g0 = wid * _RPW + k0
    in_handles[0] = (
        pltpu.async_copy(x0_hbm.at[g0, c0, rows0], xbs[0], sxs[0]),
        pltpu.async_copy(n_hbm.at[g0, c0, rows0], nbs[0], sns[0]),
    )

    for gch in range(_NCHUNKS):
        slot = gch % 2
        nslot = (gch + 1) % 2
        if gch + 1 < _NCHUNKS:
            k1, c1, rows1 = _chunk_coords(gch + 1)
            g1 = wid * _RPW + k1
            in_handles[nslot] = (
                pltpu.async_copy(x0_hbm.at[g1, c1, rows1], xbs[nslot], sxs[nslot]),
                pltpu.async_copy(n_hbm.at[g1, c1, rows1], nbs[nslot], sns[nslot]),
            )
        hx, hn = in_handles[slot]
        hx.wait()
        hn.wait()
        if out_handles[slot] is not None:
            out_handles[slot].wait()

        k, c, rows = _chunk_coords(gch)
        av, sv = coefs[k]
        xb, nb, ob = xbs[slot], nbs[slot], obs[slot]

        def _row(i, carry, xb=xb, nb=nb, ob=ob, av=av, sv=sv):
            for j in range(_W // _L):
                cols = pl.ds(j * _L, _L)
                ob[i, cols] = av * xb[i, cols] + sv * nb[i, cols]
            return carry

        lax.fori_loop(0, _RCHUNK, _row, 0)
        out_handles[slot] = pltpu.async_copy(
            ob, out_hbm.at[wid * _RPW + k, c, rows], sos[slot])

    out_handles[0].wait()
    out_handles[1].wait()


def kernel(x0, t, noise, sqrt_alphas_bar, sqrt_one_minus_alphas_bar):
    xt = _sc_qsample(
        x0,
        t.astype(jnp.int32),
        noise,
        sqrt_alphas_bar,
        sqrt_one_minus_alphas_bar,
    )
    return xt, noise


# speedup vs baseline: 1.5290x; 1.0366x over previous
import functools

import jax
import jax.numpy as jnp
from jax import lax
from jax.experimental import pallas as pl
from jax.experimental.pallas import tpu as pltpu
from jax.experimental.pallas import tpu_sc as plsc

_B, _C, _H, _W = 64, 3, 256, 256
_NC, _NS, _L = 2, 16, 16
_NW = _NC * _NS
_RPW = _B // _NW
_RCHUNK = 64
_NCH = _H // _RCHUNK
_NCHUNKS = _RPW * _C * _NCH

_mesh = plsc.VectorSubcoreMesh(core_axis_name="c", subcore_axis_name="s")


def _chunk_coords(gch):
    k = gch // (_C * _NCH)
    rem = gch % (_C * _NCH)
    return k, rem // _NCH, pl.ds((rem % _NCH) * _RCHUNK, _RCHUNK)


@functools.partial(
    pl.kernel,
    mesh=_mesh,
    out_type=jax.ShapeDtypeStruct((_B, _C, _H, _W), jnp.float32),
    scratch_types=[
        pltpu.VMEM((_B,), jnp.int32),
        pltpu.VMEM((1000,), jnp.float32),
        pltpu.VMEM((1000,), jnp.float32),
    ]
    + [pltpu.VMEM((_RCHUNK, _W), jnp.float32)] * 6
    + [pltpu.SemaphoreType.DMA] * 6,
    compiler_params=pltpu.CompilerParams(needs_layout_passes=False),
)
def _sc_qsample(x0_hbm, t_hbm, n_hbm, a_hbm, s_hbm, out_hbm,
                t_v, a_tbl, s_tbl, xb0, xb1, nb0, nb1, ob0, ob1,
                sx0, sx1, sn0, sn1, so0, so1):
    wid = lax.axis_index("s") * _NC + lax.axis_index("c")
    pltpu.sync_copy(t_hbm, t_v)
    pltpu.sync_copy(a_hbm, a_tbl)
    pltpu.sync_copy(s_hbm, s_tbl)

    coefs = []
    for k in range(_RPW):
        gidx = jnp.full((_L,), wid * _RPW + k, jnp.int32)
        tt = plsc.load_gather(t_v, [gidx])
        coefs.append((plsc.load_gather(a_tbl, [tt]),
                      plsc.load_gather(s_tbl, [tt])))

    xbs, nbs, obs = (xb0, xb1), (nb0, nb1), (ob0, ob1)
    sxs, sns, sos = (sx0, sx1), (sn0, sn1), (so0, so1)

    in_handles = [None, None]
    out_handles = [None, None]

    k0, c0, rows0 = _chunk_coords(0)
    g0 = wid * _RPW + k0
    in_handles[0] = (
        pltpu.async_copy(x0_hbm.at[g0, c0, rows0], xbs[0], sxs[0]),
        pltpu.async_copy(n_hbm.at[g0, c0, rows0], nbs[0], sns[0]),
    )

    for gch in range(_NCHUNKS):
        slot = gch % 2
        nslot = (gch + 1) % 2
        if gch + 1 < _NCHUNKS:
            k1, c1, rows1 = _chunk_coords(gch + 1)
            g1 = wid * _RPW + k1
            in_handles[nslot] = (
                pltpu.async_copy(x0_hbm.at[g1, c1, rows1], xbs[nslot], sxs[nslot]),
                pltpu.async_copy(n_hbm.at[g1, c1, rows1], nbs[nslot], sns[nslot]),
            )
        hx, hn = in_handles[slot]
        hx.wait()
        hn.wait()
        if out_handles[slot] is not None:
            out_handles[slot].wait()

        k, c, rows = _chunk_coords(gch)
        av, sv = coefs[k]
        xb, nb, ob = xbs[slot], nbs[slot], obs[slot]

        @plsc.parallel_loop(0, _RCHUNK * (_W // _L), unroll=4)
        def _grp(f, xb=xb, nb=nb, ob=ob, av=av, sv=sv):
            i = f // (_W // _L)
            cols = pl.ds((f % (_W // _L)) * _L, _L)
            ob[i, cols] = av * xb[i, cols] + sv * nb[i, cols]
        out_handles[slot] = pltpu.async_copy(
            ob, out_hbm.at[wid * _RPW + k, c, rows], sos[slot])

    out_handles[0].wait()
    out_handles[1].wait()


def kernel(x0, t, noise, sqrt_alphas_bar, sqrt_one_minus_alphas_bar):
    xt = _sc_qsample(
        x0,
        t.astype(jnp.int32),
        noise,
        sqrt_alphas_bar,
        sqrt_one_minus_alphas_bar,
    )
    return xt, noise


# speedup vs baseline: 1.5408x; 1.0077x over previous
import functools

import jax
import jax.numpy as jnp
from jax import lax
from jax.experimental import pallas as pl
from jax.experimental.pallas import tpu as pltpu
from jax.experimental.pallas import tpu_sc as plsc

_B, _C, _H, _W = 64, 3, 256, 256
_R = _C * _H
_NC, _NS, _L = 2, 16, 16
_NW = _NC * _NS
_RPW = _B // _NW
_RCHUNK = 96
_NCH = _R // _RCHUNK
_NCHUNKS = _RPW * _NCH

_mesh = plsc.VectorSubcoreMesh(core_axis_name="c", subcore_axis_name="s")


def _coords(gch):
    return gch // _NCH, pl.ds((gch % _NCH) * _RCHUNK, _RCHUNK)


@functools.partial(
    pl.kernel,
    mesh=_mesh,
    out_type=jax.ShapeDtypeStruct((_B, _R, _W), jnp.float32),
    scratch_types=[
        pltpu.VMEM((_B,), jnp.int32),
        pltpu.VMEM((1000,), jnp.float32),
        pltpu.VMEM((1000,), jnp.float32),
    ]
    + [pltpu.VMEM((_RCHUNK, _W), jnp.float32)] * 5
    + [pltpu.SemaphoreType.DMA] * 8,
    compiler_params=pltpu.CompilerParams(needs_layout_passes=False),
)
def _sc_qsample(x0_hbm, t_hbm, n_hbm, a_hbm, s_hbm, out_hbm,
                t_v, a_tbl, s_tbl, xb0, xb1, xb2, nb0, nb1,
                sx0, sx1, sx2, sn0, sn1, so0, so1, so2):
    wid = lax.axis_index("s") * _NC + lax.axis_index("c")
    pltpu.sync_copy(t_hbm, t_v)
    pltpu.sync_copy(a_hbm, a_tbl)
    pltpu.sync_copy(s_hbm, s_tbl)

    coefs = []
    for k in range(_RPW):
        gidx = jnp.full((_L,), wid * _RPW + k, jnp.int32)
        tt = plsc.load_gather(t_v, [gidx])
        coefs.append((plsc.load_gather(a_tbl, [tt]),
                      plsc.load_gather(s_tbl, [tt])))

    xbs, nbs = (xb0, xb1, xb2), (nb0, nb1)
    sxs, sns, sos = (sx0, sx1, sx2), (sn0, sn1), (so0, so1, so2)

    in_x = [None, None, None]
    in_n = [None, None]
    out_h = [None, None, None]

    for p in range(2):
        k, rows = _coords(p)
        g = wid * _RPW + k
        in_x[p] = pltpu.async_copy(x0_hbm.at[g, rows], xbs[p], sxs[p])
        in_n[p] = pltpu.async_copy(n_hbm.at[g, rows], nbs[p], sns[p])

    for gch in range(_NCHUNKS):
        xs, ns = gch % 3, gch % 2
        if gch + 2 < _NCHUNKS:
            nxs, nns = (gch + 2) % 3, gch % 2
            k2, rows2 = _coords(gch + 2)
            g2 = wid * _RPW + k2
            if out_h[nxs] is not None:
                out_h[nxs].wait()
            in_x[nxs] = pltpu.async_copy(x0_hbm.at[g2, rows2], xbs[nxs], sxs[nxs])
        in_x[xs].wait()
        in_n[ns].wait()

        k, rows = _coords(gch)
        av, sv = coefs[k]
        xb, nb = xbs[xs], nbs[ns]

        @plsc.parallel_loop(0, _RCHUNK * (_W // _L), unroll=4)
        def _grp(f, xb=xb, nb=nb, av=av, sv=sv):
            i = f // (_W // _L)
            cols = pl.ds((f % (_W // _L)) * _L, _L)
            xb[i, cols] = av * xb[i, cols] + sv * nb[i, cols]

        out_h[xs] = pltpu.async_copy(xb, out_hbm.at[wid * _RPW + k, rows], sos[xs])
        if gch + 2 < _NCHUNKS:
            k2, rows2 = _coords(gch + 2)
            g2 = wid * _RPW + k2
            in_n[ns] = pltpu.async_copy(n_hbm.at[g2, rows2], nbs[ns], sns[ns])

    for h in out_h:
        if h is not None:
            h.wait()


def kernel(x0, t, noise, sqrt_alphas_bar, sqrt_one_minus_alphas_bar):
    xt = _sc_qsample(
        x0.reshape(_B, _R, _W),
        t.astype(jnp.int32),
        noise.reshape(_B, _R, _W),
        sqrt_alphas_bar,
        sqrt_one_minus_alphas_bar,
    )
    return xt.reshape(_B, _C, _H, _W), noise


# speedup vs baseline: 1.6948x; 1.1000x over previous
import functools

import jax
import jax.numpy as jnp
from jax import lax
from jax.experimental import pallas as pl
from jax.experimental.pallas import tpu as pltpu
from jax.experimental.pallas import tpu_sc as plsc

_B = 64
_T = 1000
_L = 16
_NC = 2
_BB = 8

_mesh = plsc.VectorSubcoreMesh(core_axis_name="c", subcore_axis_name="s")


@functools.partial(
    pl.kernel,
    mesh=_mesh,
    out_type=(
        jax.ShapeDtypeStruct((_B,), jnp.float32),
        jax.ShapeDtypeStruct((_B,), jnp.float32),
    ),
    scratch_types=[
        pltpu.VMEM((_B,), jnp.int32),
        pltpu.VMEM((_T,), jnp.float32),
        pltpu.VMEM((_T,), jnp.float32),
        pltpu.VMEM((_B,), jnp.float32),
        pltpu.VMEM((_B,), jnp.float32),
    ],
    compiler_params=pltpu.CompilerParams(needs_layout_passes=False),
)
def _sc_gather_coefs(t_hbm, a_hbm, s_hbm, a_out, s_out,
                     t_v, a_tbl, s_tbl, a_v, s_v):
    wid = lax.axis_index("s") * _NC + lax.axis_index("c")
    @pl.when(wid == 0)
    def _():
        pltpu.sync_copy(t_hbm, t_v)
        pltpu.sync_copy(a_hbm, a_tbl)
        pltpu.sync_copy(s_hbm, s_tbl)
        for j in range(_B // _L):
            idx = jax.lax.iota(jnp.int32, _L) + j * _L
            tt = plsc.load_gather(t_v, [idx])
            a_v[pl.ds(j * _L, _L)] = plsc.load_gather(a_tbl, [tt])
            s_v[pl.ds(j * _L, _L)] = plsc.load_gather(s_tbl, [tt])
        pltpu.sync_copy(a_v, a_out)
        pltpu.sync_copy(s_v, s_out)


def _qsample_body(a_ref, s_ref, x0_ref, n_ref, xt_ref):
    i = pl.program_id(0)
    for k in range(_BB):
        a = a_ref[0, i * _BB + k]
        s = s_ref[0, i * _BB + k]
        xt_ref[k] = a * x0_ref[k] + s * n_ref[k]


def kernel(x0, t, noise, sqrt_alphas_bar, sqrt_one_minus_alphas_bar):
    b, c, h, w = x0.shape
    a_coef, s_coef = _sc_gather_coefs(
        t.astype(jnp.int32), sqrt_alphas_bar, sqrt_one_minus_alphas_bar)
    xt = pl.pallas_call(
        _qsample_body,
        grid=(b // _BB,),
        in_specs=[
            pl.BlockSpec(memory_space=pltpu.SMEM),
            pl.BlockSpec(memory_space=pltpu.SMEM),
            pl.BlockSpec((_BB, c, h, w), lambda i: (i, 0, 0, 0)),
            pl.BlockSpec((_BB, c, h, w), lambda i: (i, 0, 0, 0)),
        ],
        out_specs=pl.BlockSpec((_BB, c, h, w), lambda i: (i, 0, 0, 0)),
        out_shape=jax.ShapeDtypeStruct((b, c, h, w), jnp.float32),
        compiler_params=pltpu.CompilerParams(
            dimension_semantics=("parallel",),
        ),
    )(
        a_coef.reshape(1, b),
        s_coef.reshape(1, b),
        x0,
        noise,
    )
    return xt, noise


# speedup vs baseline: 1.7139x; 1.0113x over previous
import functools

import jax
import jax.numpy as jnp
from jax import lax
from jax.experimental import pallas as pl
from jax.experimental.pallas import tpu as pltpu
from jax.experimental.pallas import tpu_sc as plsc

_B = 64
_T = 1000
_L = 16
_NC = 2
_BB = 8

_mesh = plsc.VectorSubcoreMesh(core_axis_name="c", subcore_axis_name="s")


@functools.partial(
    pl.kernel,
    mesh=_mesh,
    out_type=(
        jax.ShapeDtypeStruct((_B,), jnp.float32),
        jax.ShapeDtypeStruct((_B,), jnp.float32),
    ),
    scratch_types=[
        pltpu.VMEM((_B,), jnp.int32),
        pltpu.VMEM((_T,), jnp.float32),
        pltpu.VMEM((_T,), jnp.float32),
        pltpu.VMEM((_B,), jnp.float32),
        pltpu.VMEM((_B,), jnp.float32),
        pltpu.SemaphoreType.DMA,
        pltpu.SemaphoreType.DMA,
        pltpu.SemaphoreType.DMA,
    ],
    compiler_params=pltpu.CompilerParams(needs_layout_passes=False),
)
def _sc_gather_coefs(t_hbm, a_hbm, s_hbm, a_out, s_out,
                     t_v, a_tbl, s_tbl, a_v, s_v, sem_t, sem_a, sem_s):
    wid = lax.axis_index("s") * _NC + lax.axis_index("c")
    @pl.when(wid == 0)
    def _():
        ht = pltpu.async_copy(t_hbm, t_v, sem_t)
        ha = pltpu.async_copy(a_hbm, a_tbl, sem_a)
        hs = pltpu.async_copy(s_hbm, s_tbl, sem_s)
        ht.wait()
        ha.wait()
        hs.wait()
        for j in range(_B // _L):
            idx = jax.lax.iota(jnp.int32, _L) + j * _L
            tt = plsc.load_gather(t_v, [idx])
            a_v[pl.ds(j * _L, _L)] = plsc.load_gather(a_tbl, [tt])
            s_v[pl.ds(j * _L, _L)] = plsc.load_gather(s_tbl, [tt])
        ha2 = pltpu.async_copy(a_v, a_out, sem_a)
        hs2 = pltpu.async_copy(s_v, s_out, sem_s)
        ha2.wait()
        hs2.wait()


def _qsample_body(a_ref, s_ref, x0_ref, n_ref, xt_ref):
    i = pl.program_id(0)
    for k in range(_BB):
        a = a_ref[0, i * _BB + k]
        s = s_ref[0, i * _BB + k]
        xt_ref[k] = a * x0_ref[k] + s * n_ref[k]


def kernel(x0, t, noise, sqrt_alphas_bar, sqrt_one_minus_alphas_bar):
    b, c, h, w = x0.shape
    a_coef, s_coef = _sc_gather_coefs(
        t.astype(jnp.int32), sqrt_alphas_bar, sqrt_one_minus_alphas_bar)
    xt = pl.pallas_call(
        _qsample_body,
        grid=(b // _BB,),
        in_specs=[
            pl.BlockSpec(memory_space=pltpu.SMEM),
            pl.BlockSpec(memory_space=pltpu.SMEM),
            pl.BlockSpec((_BB, c, h, w), lambda i: (i, 0, 0, 0)),
            pl.BlockSpec((_BB, c, h, w), lambda i: (i, 0, 0, 0)),
        ],
        out_specs=pl.BlockSpec((_BB, c, h, w), lambda i: (i, 0, 0, 0)),
        out_shape=jax.ShapeDtypeStruct((b, c, h, w), jnp.float32),
        compiler_params=pltpu.CompilerParams(
            dimension_semantics=("parallel",),
        ),
    )(
        a_coef.reshape(1, b),
        s_coef.reshape(1, b),
        x0,
        noise,
    )
    return xt, noise


# speedup vs baseline: 1.7375x; 1.0137x over previous
import functools

import jax
import jax.numpy as jnp
from jax import lax
from jax.experimental import pallas as pl
from jax.experimental.pallas import tpu as pltpu
from jax.experimental.pallas import tpu_sc as plsc

_B = 64
_T = 1000
_L = 16
_NC = 2
_BB = 8

_mesh = plsc.VectorSubcoreMesh(core_axis_name="c", subcore_axis_name="s", num_cores=1, num_subcores=1)


@functools.partial(
    pl.kernel,
    mesh=_mesh,
    out_type=(
        jax.ShapeDtypeStruct((_B,), jnp.float32),
        jax.ShapeDtypeStruct((_B,), jnp.float32),
    ),
    scratch_types=[
        pltpu.VMEM((_B,), jnp.int32),
        pltpu.VMEM((_T,), jnp.float32),
        pltpu.VMEM((_T,), jnp.float32),
        pltpu.VMEM((_B,), jnp.float32),
        pltpu.VMEM((_B,), jnp.float32),
        pltpu.SemaphoreType.DMA,
        pltpu.SemaphoreType.DMA,
        pltpu.SemaphoreType.DMA,
    ],
    compiler_params=pltpu.CompilerParams(needs_layout_passes=False),
)
def _sc_gather_coefs(t_hbm, a_hbm, s_hbm, a_out, s_out,
                     t_v, a_tbl, s_tbl, a_v, s_v, sem_t, sem_a, sem_s):
    wid = lax.axis_index("s") * _NC + lax.axis_index("c")
    @pl.when(wid == 0)
    def _():
        ht = pltpu.async_copy(t_hbm, t_v, sem_t)
        ha = pltpu.async_copy(a_hbm, a_tbl, sem_a)
        hs = pltpu.async_copy(s_hbm, s_tbl, sem_s)
        ht.wait()
        ha.wait()
        hs.wait()
        for j in range(_B // _L):
            idx = jax.lax.iota(jnp.int32, _L) + j * _L
            tt = plsc.load_gather(t_v, [idx])
            a_v[pl.ds(j * _L, _L)] = plsc.load_gather(a_tbl, [tt])
            s_v[pl.ds(j * _L, _L)] = plsc.load_gather(s_tbl, [tt])
        ha2 = pltpu.async_copy(a_v, a_out, sem_a)
        hs2 = pltpu.async_copy(s_v, s_out, sem_s)
        ha2.wait()
        hs2.wait()


def _qsample_body(a_ref, s_ref, x0_ref, n_ref, xt_ref):
    i = pl.program_id(0)
    for k in range(_BB):
        a = a_ref[0, i * _BB + k]
        s = s_ref[0, i * _BB + k]
        xt_ref[k] = a * x0_ref[k] + s * n_ref[k]


def kernel(x0, t, noise, sqrt_alphas_bar, sqrt_one_minus_alphas_bar):
    b, c, h, w = x0.shape
    a_coef, s_coef = _sc_gather_coefs(
        t.astype(jnp.int32), sqrt_alphas_bar, sqrt_one_minus_alphas_bar)
    xt = pl.pallas_call(
        _qsample_body,
        grid=(b // _BB,),
        in_specs=[
            pl.BlockSpec(memory_space=pltpu.SMEM),
            pl.BlockSpec(memory_space=pltpu.SMEM),
            pl.BlockSpec((_BB, c, h, w), lambda i: (i, 0, 0, 0)),
            pl.BlockSpec((_BB, c, h, w), lambda i: (i, 0, 0, 0)),
        ],
        out_specs=pl.BlockSpec((_BB, c, h, w), lambda i: (i, 0, 0, 0)),
        out_shape=jax.ShapeDtypeStruct((b, c, h, w), jnp.float32),
        compiler_params=pltpu.CompilerParams(
            dimension_semantics=("parallel",),
        ),
    )(
        a_coef.reshape(1, b),
        s_coef.reshape(1, b),
        x0,
        noise,
    )
    return xt, noise
